# baseline (speedup 1.0000x reference)
import functools

import jax
import jax.numpy as jnp
from jax import lax
from jax.experimental import pallas as pl
from jax.experimental.pallas import tpu as pltpu
from jax.experimental.pallas import tpu_sc as plsc

NC = 2
NS = 16
K = 128
NBUF = 2
PAD = 8


@functools.lru_cache(maxsize=None)
def _make_sc_agg(N, D, R, EP):
    assert R % NC == 0
    rel_per_core = R // NC
    e_per_tile = EP // NS
    assert e_per_tile % K == 0
    n_chunks = e_per_tile // K
    assert n_chunks % NBUF == 0
    rpt = (N // NS) // 8 * 8
    tail = N - NS * rpt
    assert tail % 8 == 0 and tail // 8 <= NS
    n_z128 = rpt // K
    z_rem = rpt - n_z128 * K

    mesh = plsc.VectorSubcoreMesh(core_axis_name="c", subcore_axis_name="s")

    scratch = (
        [pltpu.VMEM((n_chunks, K), jnp.int32),
         pltpu.VMEM((n_chunks, K), jnp.int32)]
        + [pltpu.VMEM((K, D), jnp.float32) for _ in range(NBUF)]
        + [pltpu.VMEM_SHARED((N, D), jnp.float32)]
        + [pltpu.SemaphoreType.DMA for _ in range(NBUF)]
    )

    @functools.partial(
        pl.kernel,
        mesh=mesh,
        out_type=jax.ShapeDtypeStruct((R, N, D), jnp.float32),
        scratch_types=scratch,
    )
    def sc_agg(x_hbm, ei_hbm, out_hbm, src_v, dst_v, *rest):
        rows = rest[:NBUF]
        acc_sh = rest[NBUF]
        sems = rest[NBUF + 1:]
        c = lax.axis_index("c")
        s = lax.axis_index("s")
        row0 = s * rpt
        trow = NS * rpt + s * 8

        for phase in range(rel_per_core):
            r = c * rel_per_core + phase

            def _zrow(i, carry):
                for j in range(D // 16):
                    rows[0][i, pl.ds(j * 16, 16)] = jnp.zeros((16,),
                                                              jnp.float32)
                return carry
            lax.fori_loop(0, K, _zrow, 0)
            for z in range(n_z128):
                pltpu.sync_copy(rows[0], acc_sh.at[pl.ds(row0 + z * K, K)])
            if z_rem:
                pltpu.sync_copy(rows[0].at[pl.ds(0, z_rem)],
                                acc_sh.at[pl.ds(row0 + n_z128 * K, z_rem)])
            if tail:
                @pl.when(s < tail // 8)
                def _():
                    pltpu.sync_copy(rows[0].at[pl.ds(0, 8)],
                                    acc_sh.at[pl.ds(trow, 8)])

            src_row0 = ((r * 2 + 0) * NS + s) * n_chunks
            dst_row0 = ((r * 2 + 1) * NS + s) * n_chunks
            pltpu.sync_copy(ei_hbm.at[pl.ds(src_row0, n_chunks)], src_v)
            pltpu.sync_copy(ei_hbm.at[pl.ds(dst_row0, n_chunks)], dst_v)
            plsc.subcore_barrier()

            for b in range(NBUF):
                pltpu.async_copy(x_hbm.at[src_v.at[b]], rows[b], sems[b])

            def _group(g, carry):
                for b in range(NBUF):
                    j = g * NBUF + b
                    pltpu.make_async_copy(x_hbm.at[src_v.at[j]], rows[b],
                                          sems[b]).wait()
                    pltpu.sync_copy(rows[b], acc_sh.at[dst_v.at[j]], add=True)
                    jn = j + NBUF

                    @pl.when(jn < n_chunks)
                    def _():
                        pltpu.async_copy(x_hbm.at[src_v.at[jn]], rows[b],
                                         sems[b])
                return carry
            lax.fori_loop(0, n_chunks // NBUF, _group, 0)
            plsc.subcore_barrier()

            pltpu.sync_copy(acc_sh.at[pl.ds(row0, rpt)],
                            out_hbm.at[r, pl.ds(row0, rpt)])
            if tail:
                @pl.when(s < tail // 8)
                def _():
                    pltpu.sync_copy(acc_sh.at[pl.ds(trow, 8)],
                                    out_hbm.at[r, pl.ds(trow, 8)])

    return sc_agg


@functools.lru_cache(maxsize=None)
def _make_tc_mm(N, D, Dout, R, bm=1000):
    grid = N // bm

    def _mm_body(x_ref, a_ref, w_ref, ws_ref, o_ref):
        acc = jnp.dot(x_ref[...], ws_ref[...],
                      preferred_element_type=jnp.float32)
        for r in range(R):
            acc = acc + jnp.dot(a_ref[r], w_ref[r],
                                preferred_element_type=jnp.float32)
        o_ref[...] = acc

    return pl.pallas_call(
        _mm_body,
        grid=(grid,),
        in_specs=[
            pl.BlockSpec((bm, D), lambda i: (i, 0)),
            pl.BlockSpec((R, bm, D), lambda i: (0, i, 0)),
            pl.BlockSpec((R, D, Dout), lambda i: (0, 0, 0)),
            pl.BlockSpec((D, Dout), lambda i: (0, 0)),
        ],
        out_specs=pl.BlockSpec((bm, Dout), lambda i: (i, 0)),
        out_shape=jax.ShapeDtypeStruct((N, Dout), jnp.float32),
    )


def kernel(x, edge_index, W, W_self):
    B, N, D = x.shape
    R, _, E = edge_index.shape
    Dout = W_self.shape[1]
    x2 = x.reshape(N, D)

    ept = -(-E // (NS * K)) * K
    EP = ept * NS
    ei = edge_index.reshape(R, 2, NS, E // NS)
    pad_val = jnp.array([N, 0], jnp.int32).reshape(1, 2, 1, 1)
    pad_blk = jnp.broadcast_to(pad_val, (R, 2, NS, ept - E // NS))
    ei = jnp.concatenate([ei, pad_blk], axis=-1)
    ei = ei.reshape(R * 2 * NS * (ept // K), K)
    xp = jnp.concatenate([x2, jnp.zeros((PAD, D), jnp.float32)])

    agg = _make_sc_agg(N, D, R, EP)(xp, ei)
    out = _make_tc_mm(N, D, Dout, R)(x2, agg, W, W_self)
    return out.reshape(B, N, Dout)
